# Initial kernel scaffold; baseline (speedup 1.0000x reference)
import functools

import jax
import jax.numpy as jnp
from jax import lax
from jax.experimental import pallas as pl
from jax.experimental.pallas import tpu as pltpu
from jax.experimental.pallas import tpu_sc as plsc

N = 10000
TWO_N = 2 * N
E = 320000
D = 128
NC, NS = 2, 16
CHUNK = 128
NCHUNK = (E // NS + CHUNK - 1) // CHUNK
EPT = NCHUNK * CHUNK
PADE = NS * EPT
DUMMY = N
ACC_ROWS = 10240
ZROWS = ACC_ROWS // NS
OROWS = N // NS

_mesh = plsc.VectorSubcoreMesh(core_axis_name="c", subcore_axis_name="s",
                               num_cores=NC, num_subcores=NS)


@functools.partial(
    pl.kernel,
    out_type=jax.ShapeDtypeStruct((TWO_N, 16), jnp.float32),
    mesh=_mesh,
    scratch_types=[
        pltpu.VMEM((NCHUNK, CHUNK), jnp.int32),
        pltpu.VMEM((CHUNK, 16), jnp.float32),
        pltpu.VMEM_SHARED((ACC_ROWS, 16), jnp.float32),
    ],
)
def _deg_kernel(ridx_hbm, ones_hbm, zeros_hbm, deg_out, ridx_v, ones_v, dacc):
    c = lax.axis_index("c")
    s = lax.axis_index("s")
    pltpu.sync_copy(zeros_hbm, dacc.at[pl.ds(s * ZROWS, ZROWS)])
    pltpu.sync_copy(ridx_hbm.at[c, s], ridx_v)
    pltpu.sync_copy(ones_hbm, ones_v)
    plsc.subcore_barrier()

    def body(j, carry):
        pltpu.sync_copy(ones_v, dacc.at[ridx_v.at[j]], add=True)
        return carry

    lax.fori_loop(0, NCHUNK, body, 0)
    plsc.subcore_barrier()
    pltpu.sync_copy(dacc.at[pl.ds(s * OROWS, OROWS)],
                    deg_out.at[pl.ds(c * N + s * OROWS, OROWS)])


@functools.partial(
    pl.kernel,
    out_type=jax.ShapeDtypeStruct((TWO_N, D), jnp.float32),
    mesh=_mesh,
    scratch_types=[
        pltpu.VMEM((NCHUNK, CHUNK), jnp.int32),
        pltpu.VMEM((NCHUNK, CHUNK), jnp.int32),
        pltpu.VMEM((CHUNK, D), jnp.float32),
        pltpu.VMEM_SHARED((ACC_ROWS, D), jnp.float32),
        pltpu.SemaphoreType.DMA,
    ],
)
def _prop_kernel(g_hbm, gidx_hbm, ridx_hbm, zeros_hbm, u_out,
                 gidx_v, ridx_v, rows_v, acc, sem):
    c = lax.axis_index("c")
    s = lax.axis_index("s")
    pltpu.sync_copy(zeros_hbm, acc.at[pl.ds(s * ZROWS, ZROWS)])
    pltpu.sync_copy(gidx_hbm.at[c, s], gidx_v)
    pltpu.sync_copy(ridx_hbm.at[c, s], ridx_v)
    plsc.subcore_barrier()

    def body(j, carry):
        pltpu.async_copy(g_hbm.at[gidx_v.at[j]], rows_v, sem).wait()
        pltpu.sync_copy(rows_v, acc.at[ridx_v.at[j]], add=True)
        return carry

    lax.fori_loop(0, NCHUNK, body, 0)
    plsc.subcore_barrier()
    pltpu.sync_copy(acc.at[pl.ds(s * OROWS, OROWS)],
                    u_out.at[pl.ds(c * N + s * OROWS, OROWS)])


_TC_BLK = 2000


def _tc_call(body, n_in):
    specs = [pl.BlockSpec((_TC_BLK, 16), lambda i: (i, 0))]
    specs += [pl.BlockSpec((_TC_BLK, D), lambda i: (i, 0))] * (n_in - 1)
    return pl.pallas_call(
        body,
        grid=(TWO_N // _TC_BLK,),
        in_specs=specs,
        out_specs=pl.BlockSpec((_TC_BLK, D), lambda i: (i, 0)),
        out_shape=jax.ShapeDtypeStruct((TWO_N, D), jnp.float32),
    )


def _scale1_body(deg_ref, f_ref, o_ref):
    sc = 1.0 / (jnp.sqrt(deg_ref[:, 0:1]) + 1e-8)
    o_ref[:] = f_ref[:] * sc


def _scale2_body(deg_ref, u_ref, o_ref):
    sc = 1.0 / (jnp.sqrt(deg_ref[:, 0:1]) + 1e-8)
    o_ref[:] = u_ref[:] * (sc * sc)


def _combine_body(deg_ref, f_ref, u1_ref, u2_ref, o_ref):
    sc = 1.0 / (jnp.sqrt(deg_ref[:, 0:1]) + 1e-8)
    o_ref[:] = (f_ref[:] + (u1_ref[:] + u2_ref[:]) * sc) * (1.0 / 3.0)


def kernel(a_feature, b_feature, edge_index):
    src = edge_index[0]
    dst = edge_index[1]
    pad = PADE - E
    zpad = jnp.zeros((pad,), jnp.int32)
    dpad = jnp.full((pad,), DUMMY, jnp.int32)
    gidx = jnp.stack([jnp.concatenate([dst + N, zpad]),
                      jnp.concatenate([src, zpad])]).reshape(2, NS, NCHUNK, CHUNK)
    ridx = jnp.stack([jnp.concatenate([src, dpad]),
                      jnp.concatenate([dst, dpad])]).reshape(2, NS, NCHUNK, CHUNK)

    ones16 = jnp.ones((CHUNK, 16), jnp.float32)
    zeros16 = jnp.zeros((ZROWS, 16), jnp.float32)
    zerosd = jnp.zeros((ZROWS, D), jnp.float32)
    f0 = jnp.concatenate([a_feature, b_feature], axis=0)

    deg = _deg_kernel(ridx, ones16, zeros16)
    g1 = _tc_call(_scale1_body, 2)(deg, f0)
    u1 = _prop_kernel(g1, gidx, ridx, zerosd)
    g2 = _tc_call(_scale2_body, 2)(deg, u1)
    u2 = _prop_kernel(g2, gidx, ridx, zerosd)
    out = _tc_call(_combine_body, 4)(deg, f0, u1, u2)
    return (out[:N], out[N:])


# speedup vs baseline: 13.2860x; 13.2860x over previous
import functools

import jax
import jax.numpy as jnp
from jax import lax
from jax.experimental import pallas as pl
from jax.experimental.pallas import tpu as pltpu
from jax.experimental.pallas import tpu_sc as plsc

N = 10000
E = 320000
D = 128
NC, NS = 2, 16
CHUNK = 128
BCH = 32
NBLK = 5
NCHUNK = NBLK * BCH
EPT = NCHUNK * CHUNK
PADE = NS * EPT
P = 10240
TP = 2 * P
DUMMY = N
ZROWS = P // NS

_mesh = plsc.VectorSubcoreMesh(core_axis_name="c", subcore_axis_name="s",
                               num_cores=NC, num_subcores=NS)


@functools.partial(
    pl.kernel,
    out_type=jax.ShapeDtypeStruct((TP, 16), jnp.float32),
    mesh=_mesh,
    scratch_types=[
        pltpu.VMEM((BCH, CHUNK), jnp.int32),
        pltpu.VMEM((CHUNK, 16), jnp.float32),
        pltpu.VMEM_SHARED((P, 16), jnp.float32),
    ],
)
def _deg_kernel(ridx_hbm, ones_hbm, zeros_hbm, deg_out, ridx_v, ones_v, dacc):
    c = lax.axis_index("c")
    s = lax.axis_index("s")
    pltpu.sync_copy(zeros_hbm, dacc.at[pl.ds(s * ZROWS, ZROWS)])
    pltpu.sync_copy(ones_hbm, ones_v)
    plsc.subcore_barrier()

    def blk(b, carry):
        pltpu.sync_copy(ridx_hbm.at[c, s, b], ridx_v)

        def body(j, carry2):
            pltpu.sync_copy(ones_v, dacc.at[ridx_v.at[j]], add=True)
            return carry2

        return lax.fori_loop(0, BCH, body, carry)

    lax.fori_loop(0, NBLK, blk, 0)
    plsc.subcore_barrier()
    pltpu.sync_copy(dacc.at[pl.ds(s * ZROWS, ZROWS)],
                    deg_out.at[pl.ds(c * P + s * ZROWS, ZROWS)])


@functools.partial(
    pl.kernel,
    out_type=jax.ShapeDtypeStruct((TP, D), jnp.float32),
    mesh=_mesh,
    scratch_types=[
        pltpu.VMEM((BCH, CHUNK), jnp.int32),
        pltpu.VMEM((BCH, CHUNK), jnp.int32),
        pltpu.VMEM((CHUNK, D), jnp.float32),
        pltpu.VMEM_SHARED((P, D), jnp.float32),
        pltpu.SemaphoreType.DMA,
    ],
)
def _prop_kernel(g_hbm, gidx_hbm, ridx_hbm, zeros_hbm, u_out,
                 gidx_v, ridx_v, rows_v, acc, sem):
    c = lax.axis_index("c")
    s = lax.axis_index("s")
    pltpu.sync_copy(zeros_hbm, acc.at[pl.ds(s * ZROWS, ZROWS)])
    plsc.subcore_barrier()

    def blk(b, carry):
        pltpu.sync_copy(gidx_hbm.at[c, s, b], gidx_v)
        pltpu.sync_copy(ridx_hbm.at[c, s, b], ridx_v)

        def body(j, carry2):
            pltpu.async_copy(g_hbm.at[gidx_v.at[j]], rows_v, sem).wait()
            pltpu.sync_copy(rows_v, acc.at[ridx_v.at[j]], add=True)
            return carry2

        return lax.fori_loop(0, BCH, body, carry)

    lax.fori_loop(0, NBLK, blk, 0)
    plsc.subcore_barrier()
    pltpu.sync_copy(acc.at[pl.ds(s * ZROWS, ZROWS)],
                    u_out.at[pl.ds(c * P + s * ZROWS, ZROWS)])


_TC_BLK = 2048


def _tc_call(body, n_in):
    specs = [pl.BlockSpec((_TC_BLK, 16), lambda i: (i, 0))]
    specs += [pl.BlockSpec((_TC_BLK, D), lambda i: (i, 0))] * (n_in - 1)
    return pl.pallas_call(
        body,
        grid=(TP // _TC_BLK,),
        in_specs=specs,
        out_specs=pl.BlockSpec((_TC_BLK, D), lambda i: (i, 0)),
        out_shape=jax.ShapeDtypeStruct((TP, D), jnp.float32),
    )


def _scale1_body(deg_ref, f_ref, o_ref):
    sc = 1.0 / (jnp.sqrt(deg_ref[:, 0:1]) + 1e-8)
    o_ref[:] = f_ref[:] * sc


def _scale2_body(deg_ref, u_ref, o_ref):
    sc = 1.0 / (jnp.sqrt(deg_ref[:, 0:1]) + 1e-8)
    o_ref[:] = u_ref[:] * (sc * sc)


def _combine_body(deg_ref, f_ref, u1_ref, u2_ref, o_ref):
    sc = 1.0 / (jnp.sqrt(deg_ref[:, 0:1]) + 1e-8)
    o_ref[:] = (f_ref[:] + (u1_ref[:] + u2_ref[:]) * sc) * (1.0 / 3.0)


def kernel(a_feature, b_feature, edge_index):
    src = edge_index[0]
    dst = edge_index[1]
    pad = PADE - E
    zpad = jnp.zeros((pad,), jnp.int32)
    dpad = jnp.full((pad,), DUMMY, jnp.int32)
    gidx = jnp.stack([jnp.concatenate([dst + P, zpad]),
                      jnp.concatenate([src, zpad])]).reshape(2, NS, NBLK, BCH, CHUNK)
    ridx = jnp.stack([jnp.concatenate([src, dpad]),
                      jnp.concatenate([dst, dpad])]).reshape(2, NS, NBLK, BCH, CHUNK)

    ones16 = jnp.ones((CHUNK, 16), jnp.float32)
    zeros16 = jnp.zeros((ZROWS, 16), jnp.float32)
    zerosd = jnp.zeros((ZROWS, D), jnp.float32)
    rowpad = jnp.zeros((P - N, D), jnp.float32)
    f0 = jnp.concatenate([a_feature, rowpad, b_feature, rowpad], axis=0)

    deg = _deg_kernel(ridx, ones16, zeros16)
    g1 = _tc_call(_scale1_body, 2)(deg, f0)
    u1 = _prop_kernel(g1, gidx, ridx, zerosd)
    g2 = _tc_call(_scale2_body, 2)(deg, u1)
    u2 = _prop_kernel(g2, gidx, ridx, zerosd)
    out = _tc_call(_combine_body, 4)(deg, f0, u1, u2)
    return (out[:N], out[P:P + N])


# speedup vs baseline: 14.5821x; 1.0976x over previous
import functools

import jax
import jax.numpy as jnp
from jax import lax
from jax.experimental import pallas as pl
from jax.experimental.pallas import tpu as pltpu
from jax.experimental.pallas import tpu_sc as plsc

N = 10000
E = 320000
D = 128
NC, NS = 2, 16
CHUNK = 128
BCH = 32
NBLK = 5
NCHUNK = NBLK * BCH
EPT = NCHUNK * CHUNK
PADE = NS * EPT
P = 10240
TP = 2 * P
DUMMY = N
ZROWS = P // NS

_mesh = plsc.VectorSubcoreMesh(core_axis_name="c", subcore_axis_name="s",
                               num_cores=NC, num_subcores=NS)


@functools.partial(
    pl.kernel,
    out_type=jax.ShapeDtypeStruct((TP, 16), jnp.float32),
    mesh=_mesh,
    scratch_types=[
        pltpu.VMEM((BCH, CHUNK), jnp.int32),
        pltpu.VMEM((CHUNK, 16), jnp.float32),
        pltpu.VMEM_SHARED((P, 16), jnp.float32),
    ],
)
def _deg_kernel(ridx_hbm, ones_hbm, zeros_hbm, deg_out, ridx_v, ones_v, dacc):
    c = lax.axis_index("c")
    s = lax.axis_index("s")
    pltpu.sync_copy(zeros_hbm, dacc.at[pl.ds(s * ZROWS, ZROWS)])
    pltpu.sync_copy(ones_hbm, ones_v)
    plsc.subcore_barrier()

    def blk(b, carry):
        pltpu.sync_copy(ridx_hbm.at[c, s, b], ridx_v)

        def body(j, carry2):
            pltpu.sync_copy(ones_v, dacc.at[ridx_v.at[j]], add=True)
            return carry2

        return lax.fori_loop(0, BCH, body, carry)

    lax.fori_loop(0, NBLK, blk, 0)
    plsc.subcore_barrier()
    pltpu.sync_copy(dacc.at[pl.ds(s * ZROWS, ZROWS)],
                    deg_out.at[pl.ds(c * P + s * ZROWS, ZROWS)])


@functools.partial(
    pl.kernel,
    out_type=jax.ShapeDtypeStruct((TP, D), jnp.float32),
    mesh=_mesh,
    scratch_types=[
        pltpu.VMEM((BCH, CHUNK), jnp.int32),
        pltpu.VMEM((BCH, CHUNK), jnp.int32),
        pltpu.VMEM((CHUNK, D), jnp.float32),
        pltpu.VMEM((CHUNK, D), jnp.float32),
        pltpu.VMEM_SHARED((P, D), jnp.float32),
        pltpu.SemaphoreType.DMA,
        pltpu.SemaphoreType.DMA,
    ],
)
def _prop_kernel(g_hbm, gidx_hbm, ridx_hbm, zeros_hbm, u_out,
                 gidx_v, ridx_v, rows0_v, rows1_v, acc, sem0, sem1):
    c = lax.axis_index("c")
    s = lax.axis_index("s")
    pltpu.sync_copy(zeros_hbm, acc.at[pl.ds(s * ZROWS, ZROWS)])
    plsc.subcore_barrier()

    def blk(b, carry):
        pltpu.sync_copy(gidx_hbm.at[c, s, b], gidx_v)
        pltpu.sync_copy(ridx_hbm.at[c, s, b], ridx_v)
        pltpu.async_copy(g_hbm.at[gidx_v.at[0]], rows0_v, sem0)

        def body(i, carry2):
            pltpu.async_copy(g_hbm.at[gidx_v.at[2 * i + 1]], rows1_v, sem1)
            pltpu.make_async_copy(g_hbm.at[gidx_v.at[2 * i]],
                                  rows0_v, sem0).wait()
            pltpu.sync_copy(rows0_v, acc.at[ridx_v.at[2 * i]], add=True)

            @pl.when(i < BCH // 2 - 1)
            def _():
                pltpu.async_copy(g_hbm.at[gidx_v.at[2 * i + 2]], rows0_v, sem0)

            pltpu.make_async_copy(g_hbm.at[gidx_v.at[2 * i + 1]],
                                  rows1_v, sem1).wait()
            pltpu.sync_copy(rows1_v, acc.at[ridx_v.at[2 * i + 1]], add=True)
            return carry2

        return lax.fori_loop(0, BCH // 2, body, carry)

    lax.fori_loop(0, NBLK, blk, 0)
    plsc.subcore_barrier()
    pltpu.sync_copy(acc.at[pl.ds(s * ZROWS, ZROWS)],
                    u_out.at[pl.ds(c * P + s * ZROWS, ZROWS)])


_TC_BLK = 2048


def _tc_call(body, n_in):
    specs = [pl.BlockSpec((_TC_BLK, 16), lambda i: (i, 0))]
    specs += [pl.BlockSpec((_TC_BLK, D), lambda i: (i, 0))] * (n_in - 1)
    return pl.pallas_call(
        body,
        grid=(TP // _TC_BLK,),
        in_specs=specs,
        out_specs=pl.BlockSpec((_TC_BLK, D), lambda i: (i, 0)),
        out_shape=jax.ShapeDtypeStruct((TP, D), jnp.float32),
    )


def _scale1_body(deg_ref, f_ref, o_ref):
    sc = 1.0 / (jnp.sqrt(deg_ref[:, 0:1]) + 1e-8)
    o_ref[:] = f_ref[:] * sc


def _scale2_body(deg_ref, u_ref, o_ref):
    sc = 1.0 / (jnp.sqrt(deg_ref[:, 0:1]) + 1e-8)
    o_ref[:] = u_ref[:] * (sc * sc)


def _combine_body(deg_ref, f_ref, u1_ref, u2_ref, o_ref):
    sc = 1.0 / (jnp.sqrt(deg_ref[:, 0:1]) + 1e-8)
    o_ref[:] = (f_ref[:] + (u1_ref[:] + u2_ref[:]) * sc) * (1.0 / 3.0)


def kernel(a_feature, b_feature, edge_index):
    src = edge_index[0]
    dst = edge_index[1]
    pad = PADE - E
    zpad = jnp.zeros((pad,), jnp.int32)
    dpad = jnp.full((pad,), DUMMY, jnp.int32)
    gidx = jnp.stack([jnp.concatenate([dst + P, zpad]),
                      jnp.concatenate([src, zpad])]).reshape(2, NS, NBLK, BCH, CHUNK)
    ridx = jnp.stack([jnp.concatenate([src, dpad]),
                      jnp.concatenate([dst, dpad])]).reshape(2, NS, NBLK, BCH, CHUNK)

    ones16 = jnp.ones((CHUNK, 16), jnp.float32)
    zeros16 = jnp.zeros((ZROWS, 16), jnp.float32)
    zerosd = jnp.zeros((ZROWS, D), jnp.float32)
    rowpad = jnp.zeros((P - N, D), jnp.float32)
    f0 = jnp.concatenate([a_feature, rowpad, b_feature, rowpad], axis=0)

    deg = _deg_kernel(ridx, ones16, zeros16)
    g1 = _tc_call(_scale1_body, 2)(deg, f0)
    u1 = _prop_kernel(g1, gidx, ridx, zerosd)
    g2 = _tc_call(_scale2_body, 2)(deg, u1)
    u2 = _prop_kernel(g2, gidx, ridx, zerosd)
    out = _tc_call(_combine_body, 4)(deg, f0, u1, u2)
    return (out[:N], out[P:P + N])
